# Initial kernel scaffold; baseline (speedup 1.0000x reference)
import jax
import jax.numpy as jnp
from jax.experimental import pallas as pl


def kernel(x, edge_index, W1, b1, W2, b2):
    raise NotImplementedError("write your pallas kernel here")


# speedup vs baseline: 50.2616x; 50.2616x over previous
import functools

import jax
import jax.numpy as jnp
from jax import lax
from jax.experimental import pallas as pl
from jax.experimental.pallas import tpu as pltpu
from jax.experimental.pallas import tpu_sc as plsc

NC = 2
NS = 16
NW = NC * NS
L = 16
CHUNK = 128

_N = 10000
_E = 320000
_D_HID = 16

N_PAD = 10112
RPS = N_PAD // NS
EPW = ((_E + NW * CHUNK - 1) // (NW * CHUNK)) * CHUNK
CHUNKS = EPW // CHUNK
E_PAD = EPW * NW


_ZC = RPS // CHUNK
_ZR = RPS - _ZC * CHUNK


def _zero_acc_slice(rows_v, acc_sh, base):
  def zbody(i, carry):
    rows_v[i] = jnp.zeros((L,), jnp.float32)
    return carry
  lax.fori_loop(0, CHUNK, zbody, 0)
  def zchunk(k, carry):
    pltpu.sync_copy(rows_v, acc_sh.at[pl.ds(base + k * CHUNK, CHUNK)])
    return carry
  lax.fori_loop(0, _ZC, zchunk, 0)
  pltpu.sync_copy(rows_v.at[pl.ds(0, _ZR)],
                  acc_sh.at[pl.ds(base + _ZC * CHUNK, _ZR)])


def _sc_agg_body(g_hbm, src_hbm, dst_hbm, out_hbm,
                 src_v, dst_v, rows_v, g_sh, acc_sh, sem):
  c = lax.axis_index("c")
  s = lax.axis_index("s")
  wid = s * NC + c
  base = s * RPS

  _zero_acc_slice(rows_v, acc_sh, base)
  pltpu.sync_copy(g_hbm.at[pl.ds(base, RPS)], g_sh.at[pl.ds(base, RPS)])
  pltpu.sync_copy(src_hbm.at[wid], src_v)
  pltpu.sync_copy(dst_hbm.at[wid], dst_v)
  plsc.subcore_barrier()

  def ebody(ci, carry):
    pltpu.async_copy(g_sh.at[src_v.at[ci]], rows_v, sem).wait()
    pltpu.sync_copy(rows_v, acc_sh.at[dst_v.at[ci]], add=True)
    return carry
  lax.fori_loop(0, CHUNKS, ebody, 0)

  plsc.subcore_barrier()
  pltpu.sync_copy(acc_sh.at[pl.ds(base, RPS)],
                  out_hbm.at[c, pl.ds(base, RPS)])


def _sc_deg_body(dst_hbm, out_hbm, dst_v, rows_v, acc_sh):
  c = lax.axis_index("c")
  s = lax.axis_index("s")
  wid = s * NC + c
  base = s * RPS

  _zero_acc_slice(rows_v, acc_sh, base)
  pltpu.sync_copy(dst_hbm.at[wid], dst_v)

  def obody(i, carry):
    rows_v[i] = jnp.ones((L,), jnp.float32)
    return carry
  lax.fori_loop(0, CHUNK, obody, 0)
  plsc.subcore_barrier()

  def ebody(ci, carry):
    pltpu.sync_copy(rows_v, acc_sh.at[dst_v.at[ci]], add=True)
    return carry
  lax.fori_loop(0, CHUNKS, ebody, 0)

  plsc.subcore_barrier()
  pltpu.sync_copy(acc_sh.at[pl.ds(base, RPS)],
                  out_hbm.at[c, pl.ds(base, RPS)])


@functools.cache
def _sc_kernels():
  mesh = plsc.VectorSubcoreMesh(
      core_axis_name="c", subcore_axis_name="s", num_cores=NC,
      num_subcores=NS)
  sc_agg = pl.kernel(
      _sc_agg_body,
      out_type=jax.ShapeDtypeStruct((NC, N_PAD, L), jnp.float32),
      mesh=mesh,
      scratch_types=[
          pltpu.VMEM((CHUNKS, CHUNK), jnp.int32),
          pltpu.VMEM((CHUNKS, CHUNK), jnp.int32),
          pltpu.VMEM((CHUNK, L), jnp.float32),
          pltpu.VMEM_SHARED((N_PAD, L), jnp.float32),
          pltpu.VMEM_SHARED((N_PAD, L), jnp.float32),
          pltpu.SemaphoreType.DMA,
      ],
  )
  sc_deg = pl.kernel(
      _sc_deg_body,
      out_type=jax.ShapeDtypeStruct((NC, N_PAD, L), jnp.float32),
      mesh=mesh,
      scratch_types=[
          pltpu.VMEM((CHUNKS, CHUNK), jnp.int32),
          pltpu.VMEM((CHUNK, L), jnp.float32),
          pltpu.VMEM_SHARED((N_PAD, L), jnp.float32),
      ],
  )
  return sc_agg, sc_deg


def _tc_hidden_body(x_ref, w1_ref, deg_ref, g1_ref):
  h = jnp.dot(x_ref[...], w1_ref[...], preferred_element_type=jnp.float32)
  deg = deg_ref[0] + deg_ref[1] + 1.0
  dinv = lax.rsqrt(deg)
  g1_ref[...] = h * dinv


def _tc_mid_body(p_ref, g1_ref, deg_ref, b1_ref, z_ref):
  deg = deg_ref[0] + deg_ref[1] + 1.0
  dinv = lax.rsqrt(deg)
  t = dinv * (p_ref[0] + p_ref[1] + g1_ref[...]) + b1_ref[...]
  z_ref[...] = jnp.maximum(t, 0.0) * dinv


def _tc_final_body(p_ref, z_ref, deg_ref, w2_ref, b2_ref, out_ref):
  deg = deg_ref[0] + deg_ref[1] + 1.0
  dinv = lax.rsqrt(deg)
  u = dinv * (p_ref[0] + p_ref[1] + z_ref[...])
  o = jnp.dot(u[:_N], w2_ref[...], preferred_element_type=jnp.float32)
  o = o + b2_ref[...]
  a = o[:, 0:1]
  b = o[:, 1:2]
  m = jnp.maximum(a, b)
  lse = m + jnp.log(jnp.exp(a - m) + jnp.exp(b - m))
  out_ref[...] = o - lse


_tc_hidden = pl.pallas_call(
    _tc_hidden_body,
    out_shape=jax.ShapeDtypeStruct((N_PAD, _D_HID), jnp.float32),
)

_tc_mid = pl.pallas_call(
    _tc_mid_body,
    out_shape=jax.ShapeDtypeStruct((N_PAD, _D_HID), jnp.float32),
)

_tc_final = pl.pallas_call(
    _tc_final_body,
    out_shape=jax.ShapeDtypeStruct((_N, 2), jnp.float32),
)


def kernel(x, edge_index, W1, b1, W2, b2):
  src = edge_index[0]
  dst = edge_index[1]
  pad_e = E_PAD - _E
  pad_ids = (_N + (jnp.arange(pad_e, dtype=jnp.int32) % L))
  src3 = jnp.concatenate([src, pad_ids]).reshape(NW, CHUNKS, CHUNK)
  dst3 = jnp.concatenate([dst, pad_ids]).reshape(NW, CHUNKS, CHUNK)
  x_pad = jnp.pad(x, ((0, N_PAD - _N), (0, 0)))

  sc_agg, sc_deg = _sc_kernels()
  deg16 = sc_deg(dst3)
  g1 = _tc_hidden(x_pad, W1, deg16)
  p1 = sc_agg(g1, src3, dst3)
  z = _tc_mid(p1, g1, deg16, b1.reshape(1, _D_HID))
  p2 = sc_agg(z, src3, dst3)
  out = _tc_final(p2, z, deg16, W2, b2.reshape(1, 2))
  return out
